# baseline (speedup 1.0000x reference)
import functools

import jax
import jax.numpy as jnp
from jax import lax
from jax.experimental import pallas as pl
from jax.experimental.pallas import tpu as pltpu
from jax.experimental.pallas import tpu_sc as plsc

_BATCH = 16384
_NUM_FEATS = 26
_EMBED_DIM = 32
_OUT_COLS = _EMBED_DIM + _NUM_FEATS - 1

_NC = 2
_NS = 16
_NW = _NC * _NS
_BPW = _BATCH // _NW
_NSLOT = 24


def _body(
    xt_hbm, tabt_hbm, out_hbm,
    idsf_v, ids_v, win_v, rows_v, feats_v, feat2_v,
    slot_sems, sem_f, sem_w,
):
    wid = lax.axis_index("s") * _NC + lax.axis_index("c")
    base = wid * _BPW
    lane = lax.iota(jnp.int32, 16)

    feat_reads = []
    for j in range(_NUM_FEATS - 1):
        feat_reads.append(
            pltpu.async_copy(
                xt_hbm.at[1 + j].at[pl.ds(base, _BPW)],
                feats_v.at[pl.ds(j * _BPW, _BPW)],
                sem_f,
            )
        )

    pltpu.sync_copy(xt_hbm.at[0].at[pl.ds(base, _BPW)], idsf_v)
    for i in range(_BPW // 16):
        ids_v[pl.ds(i * 16, 16)] = idsf_v[pl.ds(i * 16, 16)].astype(jnp.int32)

    def slot_ref(k):
        return win_v.at[pl.ds(0, _EMBED_DIM), pl.ds(k * 128, 128)]

    dummy_src = tabt_hbm.at[pl.ds(0, _EMBED_DIM), pl.ds(0, 128)]

    def fire(idv, k):
        t = pl.multiple_of((idv >> 7) * 128, 128)
        pltpu.async_copy(
            tabt_hbm.at[pl.ds(0, _EMBED_DIM), pl.ds(t, 128)],
            slot_ref(k),
            slot_sems.at[k],
        )

    def extract(idv, k, j):
        col = jnp.broadcast_to((idv & 127) + k * 128, (16,))
        v0 = plsc.load_gather(win_v, [lane, col])
        v1 = plsc.load_gather(win_v, [lane + 16, col])
        plsc.store_scatter(rows_v, [lane * _BPW + j], v0)
        plsc.store_scatter(rows_v, [(lane + 16) * _BPW + j], v1)


    v0 = [ids_v[pl.ds(16 * m, 16)] for m in range(3)]
    for u in range(48):
        slot = u % _NSLOT
        if u >= _NSLOT:
            pltpu.make_async_copy(
                dummy_src, slot_ref(slot), slot_sems.at[slot]
            ).wait()
            p = u - _NSLOT
            extract(v0[p // 16][p % 16], slot, p)
        fire(v0[u // 16][u % 16], slot)

    def iter_fn(i, _):
        j0 = i * 48
        vf = [ids_v[pl.ds(j0 + 16 * m, 16)] for m in range(3)]
        ve = [ids_v[pl.ds(j0 - 32 + 16 * m, 16)] for m in range(4)]

        for u in range(48):
            slot = u % _NSLOT
            pltpu.make_async_copy(
                dummy_src, slot_ref(slot), slot_sems.at[slot]
            ).wait()
            p = u - _NSLOT + 32
            extract(ve[p // 16][p % 16], slot, j0 + u - _NSLOT)
            fire(vf[u // 16][u % 16], slot)
        return 0

    lax.fori_loop(1, _BPW // 48, iter_fn, 0)

    j0 = (_BPW // 48) * 48
    vf = [ids_v[pl.ds(j0 + 16 * m, 16)] for m in range(2)]
    ve = [ids_v[pl.ds(j0 - 32 + 16 * m, 16)] for m in range(4)]
    for u in range(32):
        slot = u % _NSLOT
        pltpu.make_async_copy(dummy_src, slot_ref(slot), slot_sems.at[slot]).wait()
        p = u - _NSLOT + 32
        extract(ve[p // 16][p % 16], slot, j0 + u - _NSLOT)
        fire(vf[u // 16][u % 16], slot)
    vl = [ids_v[pl.ds(_BPW - 32 + 16 * m, 16)] for m in range(2)]
    for u in range(32, 32 + _NSLOT):
        slot = u % _NSLOT
        pltpu.make_async_copy(dummy_src, slot_ref(slot), slot_sems.at[slot]).wait()
        j = j0 + u - _NSLOT
        p = j - (_BPW - 32)
        extract(vl[p // 16][p % 16], slot, j)

    writes = []
    for c in range(_EMBED_DIM):
        writes.append(
            pltpu.async_copy(
                rows_v.at[pl.ds(c * _BPW, _BPW)],
                out_hbm.at[c].at[pl.ds(base, _BPW)],
                sem_w,
            )
        )

    for cp in feat_reads:
        cp.wait()
    for j in range(_NUM_FEATS - 1):
        row = _EMBED_DIM + j
        if row == 56:
            for i in range(_BPW // 16):
                feat2_v[0, pl.ds(i * 16, 16)] = feats_v[
                    pl.ds(j * _BPW + i * 16, 16)
                ]
            writes.append(
                pltpu.async_copy(
                    feat2_v, out_hbm.at[pl.ds(56, 1), pl.ds(base, _BPW)], sem_w
                )
            )
        else:
            writes.append(
                pltpu.async_copy(
                    feats_v.at[pl.ds(j * _BPW, _BPW)],
                    out_hbm.at[row].at[pl.ds(base, _BPW)],
                    sem_w,
                )
            )
    for cp in writes:
        cp.wait()


@jax.jit
def kernel(x, embed_table):
    mesh = plsc.VectorSubcoreMesh(core_axis_name="c", subcore_axis_name="s")
    f = functools.partial(
        pl.kernel,
        out_type=jax.ShapeDtypeStruct((_OUT_COLS, _BATCH), jnp.float32),
        mesh=mesh,
        scratch_types=[
            pltpu.VMEM((_BPW,), jnp.float32),
            pltpu.VMEM((_BPW,), jnp.int32),
            pltpu.VMEM((_EMBED_DIM, _NSLOT * 128), jnp.float32),
            pltpu.VMEM((_EMBED_DIM * _BPW,), jnp.float32),
            pltpu.VMEM(((_NUM_FEATS - 1) * _BPW,), jnp.float32),
            pltpu.VMEM((1, _BPW), jnp.float32),
            pltpu.SemaphoreType.DMA((_NSLOT,)),
            pltpu.SemaphoreType.DMA,
            pltpu.SemaphoreType.DMA,
        ],
        compiler_params=pltpu.CompilerParams(
            needs_layout_passes=False,
            use_tc_tiling_on_sc=True,
            disable_bounds_checks=True,
        ),
    )(_body)
    out_t = f(x.T, embed_table.T)
    return out_t.T


# speedup vs baseline: 1.0557x; 1.0557x over previous
import functools

import jax
import jax.numpy as jnp
from jax import lax
from jax.experimental import pallas as pl
from jax.experimental.pallas import tpu as pltpu
from jax.experimental.pallas import tpu_sc as plsc

_BATCH = 16384
_NUM_FEATS = 26
_EMBED_DIM = 32
_OUT_COLS = _EMBED_DIM + _NUM_FEATS - 1

_NC = 2
_NS = 16
_NW = _NC * _NS
_BPW = _BATCH // _NW
_CHUNK = 16
_NCHUNKS = _BPW // _CHUNK


def _body(
    xt_hbm, tabt_hbm, out_hbm,
    idsf_v, ids_v, win_v, rows_v, feats_v, feat2_v,
    slot_sems, sem_f, sem_w,
):
    wid = lax.axis_index("s") * _NC + lax.axis_index("c")
    base = wid * _BPW
    lane = lax.iota(jnp.int32, 16)

    feat_reads = []
    for j in range(_NUM_FEATS - 1):
        feat_reads.append(
            pltpu.async_copy(
                xt_hbm.at[1 + j].at[pl.ds(base, _BPW)],
                feats_v.at[pl.ds(j * _BPW, _BPW)],
                sem_f,
            )
        )

    pltpu.sync_copy(xt_hbm.at[0].at[pl.ds(base, _BPW)], idsf_v)
    for i in range(_BPW // 16):
        ids_v[pl.ds(i * 16, 16)] = idsf_v[pl.ds(i * 16, 16)].astype(jnp.int32)

    def slot_ref(k):
        return win_v.at[pl.ds(0, _EMBED_DIM), pl.ds(k * 128, 128)]

    dummy_src = tabt_hbm.at[pl.ds(0, _EMBED_DIM), pl.ds(0, 128)]

    def fire(idv, k):
        t = pl.multiple_of((idv >> 7) * 128, 128)
        pltpu.async_copy(
            tabt_hbm.at[pl.ds(0, _EMBED_DIM), pl.ds(t, 128)],
            slot_ref(k),
            slot_sems.at[k],
        )

    def extract(idv, k, j):
        col = jnp.broadcast_to((idv & 127) + k * 128, (16,))
        v0 = plsc.load_gather(win_v, [lane, col])
        v1 = plsc.load_gather(win_v, [lane + 16, col])
        plsc.store_scatter(rows_v, [lane * _BPW + j], v0)
        plsc.store_scatter(rows_v, [(lane + 16) * _BPW + j], v1)

    def chunk_fn(c, _):
        cm1 = lax.max(c - 1, 0)
        idvec_cur = ids_v[pl.ds(c * _CHUNK, _CHUNK)]
        idvec_prev = ids_v[pl.ds(cm1 * _CHUNK, _CHUNK)]
        for k in range(_CHUNK):
            @pl.when(c > 0)
            def _(k=k):
                pltpu.make_async_copy(
                    dummy_src, slot_ref(k), slot_sems.at[k]
                ).wait()
                extract(idvec_prev[k], k, cm1 * _CHUNK + k)
            fire(idvec_cur[k], k)
        return 0

    lax.fori_loop(0, _NCHUNKS, chunk_fn, 0)

    idvec_last = ids_v[pl.ds((_NCHUNKS - 1) * _CHUNK, _CHUNK)]
    for k in range(_CHUNK):
        pltpu.make_async_copy(dummy_src, slot_ref(k), slot_sems.at[k]).wait()
        extract(idvec_last[k], k, (_NCHUNKS - 1) * _CHUNK + k)

    writes = []
    for c in range(_EMBED_DIM):
        writes.append(
            pltpu.async_copy(
                rows_v.at[pl.ds(c * _BPW, _BPW)],
                out_hbm.at[c].at[pl.ds(base, _BPW)],
                sem_w,
            )
        )

    for cp in feat_reads:
        cp.wait()
    for j in range(_NUM_FEATS - 1):
        row = _EMBED_DIM + j
        if row == 56:
            for i in range(_BPW // 16):
                feat2_v[0, pl.ds(i * 16, 16)] = feats_v[
                    pl.ds(j * _BPW + i * 16, 16)
                ]
            writes.append(
                pltpu.async_copy(
                    feat2_v, out_hbm.at[pl.ds(56, 1), pl.ds(base, _BPW)], sem_w
                )
            )
        else:
            writes.append(
                pltpu.async_copy(
                    feats_v.at[pl.ds(j * _BPW, _BPW)],
                    out_hbm.at[row].at[pl.ds(base, _BPW)],
                    sem_w,
                )
            )
    for cp in writes:
        cp.wait()


@jax.jit
def kernel(x, embed_table):
    mesh = plsc.VectorSubcoreMesh(core_axis_name="c", subcore_axis_name="s")
    f = functools.partial(
        pl.kernel,
        out_type=jax.ShapeDtypeStruct((_OUT_COLS, _BATCH), jnp.float32),
        mesh=mesh,
        scratch_types=[
            pltpu.VMEM((_BPW,), jnp.float32),
            pltpu.VMEM((_BPW,), jnp.int32),
            pltpu.VMEM((_EMBED_DIM, _CHUNK * 128), jnp.float32),
            pltpu.VMEM((_EMBED_DIM * _BPW,), jnp.float32),
            pltpu.VMEM(((_NUM_FEATS - 1) * _BPW,), jnp.float32),
            pltpu.VMEM((1, _BPW), jnp.float32),
            pltpu.SemaphoreType.DMA((_CHUNK,)),
            pltpu.SemaphoreType.DMA,
            pltpu.SemaphoreType.DMA,
        ],
        compiler_params=pltpu.CompilerParams(
            needs_layout_passes=False,
            use_tc_tiling_on_sc=True,
            disable_bounds_checks=True,
        ),
    )(_body)
    out_t = f(x.T, embed_table.T)
    return out_t.T
